# baseline (speedup 1.0000x reference)
import functools

import jax
import jax.numpy as jnp
from jax import lax
from jax.experimental import pallas as pl
from jax.experimental.pallas import tpu as pltpu
from jax.experimental.pallas import tpu_sc as plsc

THRESHOLD = 0.5
N, H, W = 5000, 64, 64
L = 16
NC, NS = 2, 16
NW = NC * NS
BN = 256
G = -(-N // BN)
NPAD = G * BN
CHL = 256
NCH = NPAD // CHL
NG = CHL // L


def _tc_body(x_ref, cp_ref, tbl_ref):
    x = x_ref[...]
    cp_ref[...] = x
    rm = jnp.max(x, axis=1) > THRESHOLD
    cm = jnp.max(x, axis=0) > THRESHOLD
    w32 = jnp.int32(1) << lax.broadcasted_iota(jnp.int32, (32, BN), 0)

    def bits(b):
        lo = jnp.sum(jnp.where(b[:32], w32, 0), axis=0)
        hi = jnp.sum(jnp.where(b[32:], w32, 0), axis=0)
        return lo, hi

    rlo, rhi = bits(rm)
    clo, chi = bits(cm)
    z = jnp.zeros_like(rlo)
    tbl_ref[...] = jnp.stack([rlo, rhi, clo, chi, z, z, z, z], axis=0)


_tc_pass = pl.pallas_call(
    _tc_body,
    grid=(G,),
    in_specs=[pl.BlockSpec((H, W, BN), lambda g: (0, 0, g))],
    out_specs=[
        pl.BlockSpec((H, W, BN), lambda g: (0, 0, g)),
        pl.BlockSpec((8, BN), lambda g: (0, g)),
    ],
    out_shape=[
        jax.ShapeDtypeStruct((H, W, N), jnp.float32),
        jax.ShapeDtypeStruct((8, NPAD), jnp.int32),
    ],
    compiler_params=pltpu.CompilerParams(vmem_limit_bytes=50 * 1024 * 1024),
)


def _ctz(x):
    n = jnp.zeros((L,), jnp.int32)
    for shift, mask in ((16, 0xFFFF), (8, 0xFF), (4, 0xF), (2, 0x3), (1, 0x1)):
        c = (x & mask) == 0
        n = n + jnp.where(c, shift, 0)
        x = jnp.where(c, x >> shift, x)
    return n


def _fls(x):
    n = jnp.zeros((L,), jnp.int32)
    for shift, mask in ((16, -65536), (8, 0xFF00), (4, 0xF0), (2, 0xC), (1, 0x2)):
        c = (x & mask) != 0
        n = n + jnp.where(c, shift, 0)
        x = jnp.where(c, x >> shift, x)
    return n


def _minmax_idx(lo, hi, empty_min, empty_max):
    lo0 = lo == 0
    hi0 = hi == 0
    both0 = lo0 & hi0
    mn = jnp.where(both0, empty_min,
                   jnp.where(lo0, 32 + _ctz(hi), _ctz(lo)))
    mx = jnp.where(both0, empty_max,
                   jnp.where(hi0, _fls(lo), 32 + _fls(hi)))
    return mn.astype(jnp.float32), mx.astype(jnp.float32)

_mesh = plsc.VectorSubcoreMesh(core_axis_name="c", subcore_axis_name="s")


@functools.partial(
    pl.kernel,
    mesh=_mesh,
    out_type=jax.ShapeDtypeStruct((2, 2, NPAD), jnp.float32),
    scratch_types=[
        pltpu.VMEM((8, CHL), jnp.int32),
        pltpu.VMEM((2, 2, CHL), jnp.float32),
    ],
    compiler_params=pltpu.CompilerParams(needs_layout_passes=False),
)
def _sc_boxes(tbl_hbm, out_hbm, buf, obuf):
    wid = lax.axis_index("s") * NC + lax.axis_index("c")
    cid = wid

    @pl.when(cid < NCH)
    def _process():
        base = cid * CHL
        pltpu.sync_copy(tbl_hbm.at[:, pl.ds(base, CHL)], buf)

        def group_body(g, _):
            off = pl.multiple_of(g * L, L)
            rlo = buf[0, pl.ds(off, L)]
            rhi = buf[1, pl.ds(off, L)]
            clo = buf[2, pl.ds(off, L)]
            chi = buf[3, pl.ds(off, L)]
            mnr, mxr = _minmax_idx(rlo, rhi, H, -1)
            mnc, mxc = _minmax_idx(clo, chi, W, -1)
            obuf[0, 0, pl.ds(off, L)] = mnc
            obuf[0, 1, pl.ds(off, L)] = mnr
            obuf[1, 0, pl.ds(off, L)] = mxc
            obuf[1, 1, pl.ds(off, L)] = mxr
            return 0

        lax.fori_loop(0, NG, group_body, 0)
        pltpu.sync_copy(obuf, out_hbm.at[:, :, pl.ds(base, CHL)])


def kernel(masks):
    mt = jnp.transpose(masks, (1, 2, 0))
    cp, tbl = _tc_pass(mt)
    b4 = _sc_boxes(tbl)
    masks_out = jnp.transpose(cp, (2, 0, 1))
    boxes_2d = jnp.transpose(b4[:, :, :N], (2, 0, 1))
    return masks_out, boxes_2d


# speedup vs baseline: 1.0235x; 1.0235x over previous
import functools

import jax
import jax.numpy as jnp
from jax import lax
from jax.experimental import pallas as pl
from jax.experimental.pallas import tpu as pltpu
from jax.experimental.pallas import tpu_sc as plsc

THRESHOLD = 0.5
N, H, W = 5000, 64, 64
L = 16
NC, NS = 2, 16
NW = NC * NS
BN = 512
G = -(-N // BN)
NPAD = G * BN
CHL = 256
NCH = NPAD // CHL
NG = CHL // L


def _tc_body(x_ref, cp_ref, tbl_ref):
    x = x_ref[...]
    cp_ref[...] = x
    rm = jnp.max(x, axis=1) > THRESHOLD
    cm = jnp.max(x, axis=0) > THRESHOLD
    w32 = jnp.int32(1) << lax.broadcasted_iota(jnp.int32, (32, BN), 0)

    def bits(b):
        lo = jnp.sum(jnp.where(b[:32], w32, 0), axis=0)
        hi = jnp.sum(jnp.where(b[32:], w32, 0), axis=0)
        return lo, hi

    rlo, rhi = bits(rm)
    clo, chi = bits(cm)
    z = jnp.zeros_like(rlo)
    tbl_ref[...] = jnp.stack([rlo, rhi, clo, chi, z, z, z, z], axis=0)


_tc_pass = pl.pallas_call(
    _tc_body,
    grid=(G,),
    in_specs=[pl.BlockSpec((H, W, BN), lambda g: (0, 0, g))],
    out_specs=[
        pl.BlockSpec((H, W, BN), lambda g: (0, 0, g)),
        pl.BlockSpec((8, BN), lambda g: (0, g)),
    ],
    out_shape=[
        jax.ShapeDtypeStruct((H, W, N), jnp.float32),
        jax.ShapeDtypeStruct((8, NPAD), jnp.int32),
    ],
    compiler_params=pltpu.CompilerParams(vmem_limit_bytes=50 * 1024 * 1024),
)


def _ctz(x):
    n = jnp.zeros((L,), jnp.int32)
    for shift, mask in ((16, 0xFFFF), (8, 0xFF), (4, 0xF), (2, 0x3), (1, 0x1)):
        c = (x & mask) == 0
        n = n + jnp.where(c, shift, 0)
        x = jnp.where(c, x >> shift, x)
    return n


def _fls(x):
    n = jnp.zeros((L,), jnp.int32)
    for shift, mask in ((16, -65536), (8, 0xFF00), (4, 0xF0), (2, 0xC), (1, 0x2)):
        c = (x & mask) != 0
        n = n + jnp.where(c, shift, 0)
        x = jnp.where(c, x >> shift, x)
    return n


def _minmax_idx(lo, hi, empty_min, empty_max):
    lo0 = lo == 0
    hi0 = hi == 0
    both0 = lo0 & hi0
    mn = jnp.where(both0, empty_min,
                   jnp.where(lo0, 32 + _ctz(hi), _ctz(lo)))
    mx = jnp.where(both0, empty_max,
                   jnp.where(hi0, _fls(lo), 32 + _fls(hi)))
    return mn.astype(jnp.float32), mx.astype(jnp.float32)

_mesh = plsc.VectorSubcoreMesh(core_axis_name="c", subcore_axis_name="s")


@functools.partial(
    pl.kernel,
    mesh=_mesh,
    out_type=jax.ShapeDtypeStruct((2, 2, NPAD), jnp.float32),
    scratch_types=[
        pltpu.VMEM((8, CHL), jnp.int32),
        pltpu.VMEM((2, 2, CHL), jnp.float32),
    ],
    compiler_params=pltpu.CompilerParams(
        needs_layout_passes=False, skip_device_barrier=True),
)
def _sc_boxes(tbl_hbm, out_hbm, buf, obuf):
    wid = lax.axis_index("s") * NC + lax.axis_index("c")
    cid = wid

    @pl.when(cid < NCH)
    def _process():
        base = cid * CHL
        pltpu.sync_copy(tbl_hbm.at[:, pl.ds(base, CHL)], buf)

        def group_body(g, _):
            off = pl.multiple_of(g * L, L)
            rlo = buf[0, pl.ds(off, L)]
            rhi = buf[1, pl.ds(off, L)]
            clo = buf[2, pl.ds(off, L)]
            chi = buf[3, pl.ds(off, L)]
            mnr, mxr = _minmax_idx(rlo, rhi, H, -1)
            mnc, mxc = _minmax_idx(clo, chi, W, -1)
            obuf[0, 0, pl.ds(off, L)] = mnc
            obuf[0, 1, pl.ds(off, L)] = mnr
            obuf[1, 0, pl.ds(off, L)] = mxc
            obuf[1, 1, pl.ds(off, L)] = mxr
            return 0

        lax.fori_loop(0, NG, group_body, 0)
        pltpu.sync_copy(obuf, out_hbm.at[:, :, pl.ds(base, CHL)])


def kernel(masks):
    mt = jnp.transpose(masks, (1, 2, 0))
    cp, tbl = _tc_pass(mt)
    b4 = _sc_boxes(tbl)
    masks_out = jnp.transpose(cp, (2, 0, 1))
    boxes_2d = jnp.transpose(b4[:, :, :N], (2, 0, 1))
    return masks_out, boxes_2d


# speedup vs baseline: 1.0243x; 1.0008x over previous
import functools

import jax
import jax.numpy as jnp
from jax import lax
from jax.experimental import pallas as pl
from jax.experimental.pallas import tpu as pltpu
from jax.experimental.pallas import tpu_sc as plsc

THRESHOLD = 0.5
N, H, W = 5000, 64, 64
L = 16
NC, NS = 2, 16
NW = NC * NS
BN = 512
G = -(-N // BN)
NPAD = G * BN
CHL = 256
NCH = NPAD // CHL
NG = CHL // L


def _tc_body(x_ref, cp_ref, tbl_ref):
    x = x_ref[...]
    cp_ref[...] = x
    rm = jnp.max(x, axis=1) > THRESHOLD
    cm = jnp.max(x, axis=0) > THRESHOLD
    w32 = jnp.int32(1) << lax.broadcasted_iota(jnp.int32, (32, BN), 0)

    def bits(b):
        lo = jnp.sum(jnp.where(b[:32], w32, 0), axis=0)
        hi = jnp.sum(jnp.where(b[32:], w32, 0), axis=0)
        return lo, hi

    rlo, rhi = bits(rm)
    clo, chi = bits(cm)
    z = jnp.zeros_like(rlo)
    tbl_ref[...] = jnp.stack([rlo, rhi, clo, chi, z, z, z, z], axis=0)


_tc_pass = pl.pallas_call(
    _tc_body,
    grid=(G,),
    in_specs=[pl.BlockSpec((H, W, BN), lambda g: (0, 0, g))],
    out_specs=[
        pl.BlockSpec((H, W, BN), lambda g: (0, 0, g)),
        pl.BlockSpec((8, BN), lambda g: (0, g)),
    ],
    out_shape=[
        jax.ShapeDtypeStruct((H, W, N), jnp.float32),
        jax.ShapeDtypeStruct((8, NPAD), jnp.int32),
    ],
    compiler_params=pltpu.CompilerParams(vmem_limit_bytes=50 * 1024 * 1024),
)


def _ctz(x):
    n = jnp.zeros((L,), jnp.int32)
    for shift, mask in ((16, 0xFFFF), (8, 0xFF), (4, 0xF), (2, 0x3), (1, 0x1)):
        c = (x & mask) == 0
        n = n + jnp.where(c, shift, 0)
        x = jnp.where(c, x >> shift, x)
    return n


def _fls(x):
    n = jnp.zeros((L,), jnp.int32)
    for shift, mask in ((16, -65536), (8, 0xFF00), (4, 0xF0), (2, 0xC), (1, 0x2)):
        c = (x & mask) != 0
        n = n + jnp.where(c, shift, 0)
        x = jnp.where(c, x >> shift, x)
    return n


def _minmax_idx(lo, hi, empty_min, empty_max):
    lo0 = lo == 0
    hi0 = hi == 0
    both0 = lo0 & hi0
    mn = jnp.where(both0, empty_min,
                   jnp.where(lo0, 32 + _ctz(hi), _ctz(lo)))
    mx = jnp.where(both0, empty_max,
                   jnp.where(hi0, _fls(lo), 32 + _fls(hi)))
    return mn.astype(jnp.float32), mx.astype(jnp.float32)

_mesh = plsc.VectorSubcoreMesh(core_axis_name="c", subcore_axis_name="s")


@functools.partial(
    pl.kernel,
    mesh=_mesh,
    out_type=jax.ShapeDtypeStruct((2, 2, NPAD), jnp.float32),
    scratch_types=[
        pltpu.VMEM((8, CHL), jnp.int32),
        pltpu.VMEM((2, 2, CHL), jnp.float32),
    ],
    compiler_params=pltpu.CompilerParams(needs_layout_passes=False),
)
def _sc_boxes(tbl_hbm, out_hbm, buf, obuf):
    wid = lax.axis_index("s") * NC + lax.axis_index("c")
    cid = wid

    @pl.when(cid < NCH)
    def _process():
        base = cid * CHL
        pltpu.sync_copy(tbl_hbm.at[:, pl.ds(base, CHL)], buf)

        def group_body(g, _):
            off = pl.multiple_of(g * L, L)
            rlo = buf[0, pl.ds(off, L)]
            rhi = buf[1, pl.ds(off, L)]
            clo = buf[2, pl.ds(off, L)]
            chi = buf[3, pl.ds(off, L)]
            mnr, mxr = _minmax_idx(rlo, rhi, H, -1)
            mnc, mxc = _minmax_idx(clo, chi, W, -1)
            obuf[0, 0, pl.ds(off, L)] = mnc
            obuf[0, 1, pl.ds(off, L)] = mnr
            obuf[1, 0, pl.ds(off, L)] = mxc
            obuf[1, 1, pl.ds(off, L)] = mxr
            return 0

        lax.fori_loop(0, NG, group_body, 0)
        pltpu.sync_copy(obuf, out_hbm.at[:, :, pl.ds(base, CHL)])


def kernel(masks):
    mt = jnp.transpose(masks, (1, 2, 0))
    cp, tbl = _tc_pass(mt)
    b4 = _sc_boxes(tbl)
    masks_out = jnp.transpose(cp, (2, 0, 1))
    boxes_2d = jnp.transpose(b4[:, :, :N], (2, 0, 1))
    return masks_out, boxes_2d
